# Initial kernel scaffold; baseline (speedup 1.0000x reference)
import jax
import jax.numpy as jnp
from jax.experimental import pallas as pl


def kernel(x, edge_index, edge_attr, W1, b1, W2, b2, W3, b3):
    raise NotImplementedError("write your pallas kernel here")


# speedup vs baseline: 7.7335x; 7.7335x over previous
import functools

import jax
import jax.numpy as jnp
from jax import lax
from jax.experimental import pallas as pl
from jax.experimental.pallas import tpu as pltpu
from jax.experimental.pallas import tpu_sc as plsc

_N = 10000
_E = 320000
_D = 128
_DO = 40
_DOP = 128

_NC = 2
_NS = 16
_NW = _NC * _NS
_CHUNK = 128
_CPW = 80
_EPW = _CPW * _CHUNK
_EPAD = _NW * _EPW
_NP = 10240
_RPT = _NP // _NS

_mesh = plsc.VectorSubcoreMesh(core_axis_name="c", subcore_axis_name="s")

_ROW_PIECES = tuple((i * _CHUNK, _CHUNK) for i in range(_RPT // _CHUNK))


def _zero_rows(buf, d):
    zero = jnp.zeros((16,), jnp.float32)

    def body(j, _):
        for k in range(d // 16):
            buf[j, pl.ds(k * 16, 16)] = zero
        return 0

    lax.fori_loop(0, _CHUNK, body, 0)


@functools.partial(
    pl.kernel,
    out_type=jax.ShapeDtypeStruct((_NC, _NP, 16), jnp.float32),
    mesh=_mesh,
    compiler_params=pltpu.CompilerParams(needs_layout_passes=False),
    scratch_types=[
        pltpu.VMEM((_CPW, _CHUNK), jnp.int32),
        pltpu.VMEM((_EPW,), jnp.float32),
        pltpu.VMEM((_CHUNK, 16), jnp.float32),
        pltpu.VMEM_SHARED((_NP, 16), jnp.float32),
    ],
)
def _deg_kernel(dst_hbm, w_hbm, out_hbm, dst_v, w_v, buf_v, acc_sh):
    cid = lax.axis_index("c")
    sid = lax.axis_index("s")
    wid = cid * _NS + sid
    pltpu.sync_copy(dst_hbm.at[wid], dst_v)
    pltpu.sync_copy(w_hbm.at[wid], w_v)
    _zero_rows(buf_v, 16)
    for r0, nr in _ROW_PIECES:
        pltpu.sync_copy(buf_v.at[pl.ds(0, nr)],
                        acc_sh.at[pl.ds(sid * _RPT + r0, nr)])
    plsc.subcore_barrier()

    def chunk_body(c, _):
        def edge_body(j, _):
            wj = plsc.load_gather(
                w_v, [jnp.full((16,), c * _CHUNK + j, jnp.int32)])
            buf_v[j, :] = wj
            return 0
        lax.fori_loop(0, _CHUNK, edge_body, 0)
        pltpu.sync_copy(buf_v, acc_sh.at[dst_v.at[c]], add=True)
        return 0

    lax.fori_loop(0, _CPW, chunk_body, 0)
    plsc.subcore_barrier()
    for r0, nr in _ROW_PIECES:
        pltpu.sync_copy(acc_sh.at[pl.ds(sid * _RPT + r0, nr)],
                        out_hbm.at[cid, pl.ds(sid * _RPT + r0, nr)])


def _make_agg_kernel(d):
    @functools.partial(
        pl.kernel,
        out_type=jax.ShapeDtypeStruct((_NC, _NP, d), jnp.float32),
        mesh=_mesh,
        compiler_params=pltpu.CompilerParams(needs_layout_passes=False),
        scratch_types=[
            pltpu.VMEM((2, 4, _CHUNK), jnp.int32),
            pltpu.VMEM((_N,), jnp.float32),
            pltpu.VMEM((_CHUNK,), jnp.float32),
            pltpu.VMEM((2, _CHUNK, d), jnp.float32),
            pltpu.VMEM_SHARED((_NP, d), jnp.float32),
            pltpu.SemaphoreType.DMA,
            pltpu.SemaphoreType.DMA,
            pltpu.SemaphoreType.DMA,
            pltpu.SemaphoreType.DMA,
        ],
    )
    def _agg(u_hbm, dinv_hbm, eidx_hbm, out_hbm,
             idx_v, dinv_v, s_v, rows_v, acc_sh, gs0, gs1, is0, is1):
        gsems = (gs0, gs1)
        isems = (is0, is1)
        cid = lax.axis_index("c")
        sid = lax.axis_index("s")
        wid = cid * _NS + sid
        pltpu.sync_copy(dinv_hbm, dinv_v)
        _zero_rows(rows_v.at[0], d)
        for r0, nr in _ROW_PIECES:
            pltpu.sync_copy(rows_v.at[0, pl.ds(0, nr)],
                            acc_sh.at[pl.ds(sid * _RPT + r0, nr)])
        plsc.subcore_barrier()

        pltpu.async_copy(eidx_hbm.at[wid, 0], idx_v.at[0], is0)
        pltpu.async_copy(eidx_hbm.at[wid, 1], idx_v.at[1], is1)
        pltpu.make_async_copy(eidx_hbm.at[wid, 0], idx_v.at[0], is0).wait()
        pltpu.async_copy(u_hbm.at[idx_v.at[0, 0]], rows_v.at[0], gs0)

        def pair_body(i, _):
            for b in range(2):
                c = 2 * i + b
                nb = 1 - b

                @pl.when(c + 1 < _CPW)
                def _():
                    pltpu.make_async_copy(
                        eidx_hbm.at[wid, c + 1], idx_v.at[nb],
                        isems[nb]).wait()
                    pltpu.async_copy(
                        u_hbm.at[idx_v.at[nb, 0]], rows_v.at[nb], gsems[nb])

                pltpu.make_async_copy(
                    u_hbm.at[idx_v.at[b, 0]], rows_v.at[b], gsems[b]).wait()

                def scale_prep(t, _):
                    sl = pl.ds(t * 16, 16)
                    src16 = idx_v[b, 0, sl]
                    w16 = plsc.bitcast(idx_v[b, 2, sl], jnp.float32)
                    s_v[sl] = w16 * plsc.load_gather(dinv_v, [src16])
                    return 0
                lax.fori_loop(0, _CHUNK // 16, scale_prep, 0)

                def edge_body(j, _):
                    sb = plsc.load_gather(
                        s_v, [jnp.full((16,), j, jnp.int32)])
                    for k in range(d // 16):
                        sl = pl.ds(k * 16, 16)
                        rows_v[b, j, sl] = rows_v[b, j, sl] * sb
                    return 0
                lax.fori_loop(0, _CHUNK, edge_body, 0)

                pltpu.sync_copy(rows_v.at[b], acc_sh.at[idx_v.at[b, 1]],
                                add=True)

                @pl.when(c + 2 < _CPW)
                def _():
                    pltpu.async_copy(
                        eidx_hbm.at[wid, c + 2], idx_v.at[b], isems[b])
            return 0

        lax.fori_loop(0, _CPW // 2, pair_body, 0)
        plsc.subcore_barrier()
        for r0, nr in _ROW_PIECES:
            pltpu.sync_copy(acc_sh.at[pl.ds(sid * _RPT + r0, nr)],
                            out_hbm.at[cid, pl.ds(sid * _RPT + r0, nr)])

    return _agg


_agg128 = _make_agg_kernel(_D)


_RB = 1000


def _dinv_body(p0_ref, p1_ref, o_ref):
    deg = 1.0 + p0_ref[...] + p1_ref[...]
    o_ref[...] = lax.rsqrt(deg[:, 0:1])


def _mm1_body(x_ref, w_ref, o_ref):
    o_ref[...] = jnp.dot(x_ref[...], w_ref[...],
                         preferred_element_type=jnp.float32)


def _mm_mid_body(p0_ref, p1_ref, u_ref, dinv_ref, b_ref, w_ref, o_ref):
    dinv = dinv_ref[...]
    a = dinv * (p0_ref[...] + p1_ref[...] + dinv * u_ref[...]) + b_ref[...]
    h = jnp.where(a > 0, a, jnp.exp(jnp.minimum(a, 0.0)) - 1.0)
    o_ref[...] = jnp.dot(h, w_ref[...], preferred_element_type=jnp.float32)


def _final_body(p0_ref, p1_ref, u_ref, dinv_ref, b_ref, o_ref):
    dinv = dinv_ref[...]
    a = dinv * (p0_ref[...] + p1_ref[...] + dinv * u_ref[...]) + b_ref[...]
    col = lax.broadcasted_iota(jnp.int32, a.shape, 1)
    am = jnp.where(col < _DO, a, -jnp.inf)
    m = jnp.max(am, axis=1, keepdims=True)
    lse = jnp.log(jnp.sum(jnp.exp(am - m), axis=1, keepdims=True)) + m
    o_ref[...] = am - lse


def _row_spec(d):
    return pl.BlockSpec((_RB, d), lambda i: (i, 0))


def _full_spec(r, c):
    return pl.BlockSpec((r, c), lambda i: (0, 0))


def kernel(x, edge_index, edge_attr, W1, b1, W2, b2, W3, b3):
    src = edge_index[0]
    dst = edge_index[1]
    w = edge_attr

    pad = _EPAD - _E
    zpad_i = jnp.zeros((pad,), jnp.int32)
    src3 = jnp.concatenate([src, zpad_i]).reshape(_NW, _CPW, _CHUNK)
    dst3 = jnp.concatenate([dst, zpad_i]).reshape(_NW, _CPW, _CHUNK)
    w3 = jnp.concatenate([w, jnp.zeros((pad,), jnp.float32)]
                         ).reshape(_NW, _CPW, _CHUNK)
    eidx = jnp.stack([src3, dst3, lax.bitcast_convert_type(w3, jnp.int32),
                      jnp.zeros_like(src3)], axis=2)

    W3p = jnp.pad(W3, ((0, 0), (0, _DOP - _DO)))
    b3p = jnp.pad(b3, (0, _DOP - _DO)).reshape(1, _DOP)
    b1r = b1.reshape(1, _D)
    b2r = b2.reshape(1, _D)

    grid = (_N // _RB,)

    pdeg = _deg_kernel(dst3, w3.reshape(_NW, _EPW))
    u1 = pl.pallas_call(
        _mm1_body,
        grid=grid,
        in_specs=[_row_spec(_D), _full_spec(_D, _D)],
        out_specs=_row_spec(_D),
        out_shape=jax.ShapeDtypeStruct((_N, _D), jnp.float32),
    )(x, W1)

    dinv = pl.pallas_call(
        _dinv_body,
        grid=grid,
        in_specs=[_row_spec(16), _row_spec(16)],
        out_specs=_row_spec(1),
        out_shape=jax.ShapeDtypeStruct((_N, 1), jnp.float32),
    )(pdeg[0], pdeg[1])
    dinv_flat = dinv.reshape(_N)

    p1_ = _agg128(u1, dinv_flat, eidx)
    u2 = pl.pallas_call(
        _mm_mid_body,
        grid=grid,
        in_specs=[_row_spec(_D), _row_spec(_D), _row_spec(_D), _row_spec(1),
                  _full_spec(1, _D), _full_spec(_D, _D)],
        out_specs=_row_spec(_D),
        out_shape=jax.ShapeDtypeStruct((_N, _D), jnp.float32),
    )(p1_[0], p1_[1], u1, dinv, b1r, W2)

    p2_ = _agg128(u2, dinv_flat, eidx)
    u3 = pl.pallas_call(
        _mm_mid_body,
        grid=grid,
        in_specs=[_row_spec(_D), _row_spec(_D), _row_spec(_D), _row_spec(1),
                  _full_spec(1, _D), _full_spec(_D, _DOP)],
        out_specs=_row_spec(_DOP),
        out_shape=jax.ShapeDtypeStruct((_N, _DOP), jnp.float32),
    )(p2_[0], p2_[1], u2, dinv, b2r, W3p)

    p3_ = _agg128(u3, dinv_flat, eidx)
    out = pl.pallas_call(
        _final_body,
        grid=grid,
        in_specs=[_row_spec(_DOP), _row_spec(_DOP), _row_spec(_DOP),
                  _row_spec(1), _full_spec(1, _DOP)],
        out_specs=_row_spec(_DOP),
        out_shape=jax.ShapeDtypeStruct((_N, _DOP), jnp.float32),
    )(p3_[0], p3_[1], u3, dinv, b3p)
    return out[:, :_DO]


# speedup vs baseline: 7.8138x; 1.0104x over previous
import functools

import jax
import jax.numpy as jnp
from jax import lax
from jax.experimental import pallas as pl
from jax.experimental.pallas import tpu as pltpu
from jax.experimental.pallas import tpu_sc as plsc

_N = 10000
_E = 320000
_D = 128
_DO = 40
_DOP = 128

_NC = 2
_NS = 16
_NW = _NC * _NS
_CHUNK = 128
_CPW = 80
_EPW = _CPW * _CHUNK
_EPAD = _NW * _EPW
_NP = 10240
_RPT = _NP // _NS

_mesh = plsc.VectorSubcoreMesh(core_axis_name="c", subcore_axis_name="s")

_ROW_PIECES = tuple((i * _CHUNK, _CHUNK) for i in range(_RPT // _CHUNK))


def _zero_rows(buf, d):
    zero = jnp.zeros((16,), jnp.float32)

    def body(j, _):
        for k in range(d // 16):
            buf[j, pl.ds(k * 16, 16)] = zero
        return 0

    lax.fori_loop(0, _CHUNK, body, 0)


@functools.partial(
    pl.kernel,
    out_type=jax.ShapeDtypeStruct((_NC, _NP, 16), jnp.float32),
    mesh=_mesh,
    compiler_params=pltpu.CompilerParams(needs_layout_passes=False),
    scratch_types=[
        pltpu.VMEM((_CPW, _CHUNK), jnp.int32),
        pltpu.VMEM((_EPW,), jnp.float32),
        pltpu.VMEM((_CHUNK, 16), jnp.float32),
        pltpu.VMEM_SHARED((_NP, 16), jnp.float32),
    ],
)
def _deg_kernel(dst_hbm, w_hbm, out_hbm, dst_v, w_v, buf_v, acc_sh):
    cid = lax.axis_index("c")
    sid = lax.axis_index("s")
    wid = cid * _NS + sid
    pltpu.sync_copy(dst_hbm.at[wid], dst_v)
    pltpu.sync_copy(w_hbm.at[wid], w_v)
    _zero_rows(buf_v, 16)
    for r0, nr in _ROW_PIECES:
        pltpu.sync_copy(buf_v.at[pl.ds(0, nr)],
                        acc_sh.at[pl.ds(sid * _RPT + r0, nr)])
    plsc.subcore_barrier()

    def chunk_body(c, _):
        def edge_body(j, _):
            wj = plsc.load_gather(
                w_v, [jnp.full((16,), c * _CHUNK + j, jnp.int32)])
            buf_v[j, :] = wj
            return 0
        lax.fori_loop(0, _CHUNK, edge_body, 0)
        pltpu.sync_copy(buf_v, acc_sh.at[dst_v.at[c]], add=True)
        return 0

    lax.fori_loop(0, _CPW, chunk_body, 0)
    plsc.subcore_barrier()
    for r0, nr in _ROW_PIECES:
        pltpu.sync_copy(acc_sh.at[pl.ds(sid * _RPT + r0, nr)],
                        out_hbm.at[cid, pl.ds(sid * _RPT + r0, nr)])


def _make_agg_kernel(d):
    @functools.partial(
        pl.kernel,
        out_type=jax.ShapeDtypeStruct((_NC, _NP, d), jnp.float32),
        mesh=_mesh,
        compiler_params=pltpu.CompilerParams(needs_layout_passes=False),
        scratch_types=[
            pltpu.VMEM((4, 4, _CHUNK), jnp.int32),
            pltpu.VMEM((_N,), jnp.float32),
            pltpu.VMEM((_CHUNK,), jnp.float32),
            pltpu.VMEM((2, _CHUNK, d), jnp.float32),
            pltpu.VMEM_SHARED((_NP, d), jnp.float32),
            pltpu.SemaphoreType.DMA,
            pltpu.SemaphoreType.DMA,
            pltpu.SemaphoreType.DMA,
            pltpu.SemaphoreType.DMA,
            pltpu.SemaphoreType.DMA,
            pltpu.SemaphoreType.DMA,
            pltpu.SemaphoreType.DMA,
            pltpu.SemaphoreType.DMA,
        ],
    )
    def _agg(u_hbm, dinv_hbm, eidx_hbm, out_hbm,
             idx_v, dinv_v, s_v, rows_v, acc_sh,
             gs0, gs1, ss0, ss1, is0, is1, is2, is3):
        gsems = (gs0, gs1)
        ssems = (ss0, ss1)
        isems = (is0, is1, is2, is3)
        cid = lax.axis_index("c")
        sid = lax.axis_index("s")
        wid = cid * _NS + sid
        pltpu.sync_copy(dinv_hbm, dinv_v)
        _zero_rows(rows_v.at[0], d)
        for r0, nr in _ROW_PIECES:
            pltpu.sync_copy(rows_v.at[0, pl.ds(0, nr)],
                            acc_sh.at[pl.ds(sid * _RPT + r0, nr)])
        plsc.subcore_barrier()

        for q in range(2):
            pltpu.async_copy(eidx_hbm.at[wid, q], idx_v.at[q], isems[q])
        pltpu.make_async_copy(eidx_hbm.at[wid, 0], idx_v.at[0], is0).wait()
        pltpu.async_copy(u_hbm.at[idx_v.at[0, 0]], rows_v.at[0], gs0)

        def quad_body(i, _):
            for b4 in range(4):
                c = 4 * i + b4
                b = b4 % 2
                nb = 1 - b
                q = b4
                nq = (b4 + 1) % 4

                @pl.when(c >= 1)
                def _():
                    pltpu.make_async_copy(
                        rows_v.at[nb],
                        acc_sh.at[idx_v.at[(q + 3) % 4, 1]],
                        ssems[nb]).wait()

                @pl.when(c + 1 < _CPW)
                def _():
                    pltpu.make_async_copy(
                        eidx_hbm.at[wid, c + 1], idx_v.at[nq],
                        isems[nq]).wait()
                    pltpu.async_copy(
                        u_hbm.at[idx_v.at[nq, 0]], rows_v.at[nb], gsems[nb])

                pltpu.make_async_copy(
                    u_hbm.at[idx_v.at[q, 0]], rows_v.at[b], gsems[b]).wait()

                @plsc.parallel_loop(0, _CHUNK // 16, unroll=2)
                def _(t):
                    sl = pl.ds(t * 16, 16)
                    src16 = idx_v[q, 0, sl]
                    w16 = plsc.bitcast(idx_v[q, 2, sl], jnp.float32)
                    s_v[sl] = w16 * plsc.load_gather(dinv_v, [src16])

                @plsc.parallel_loop(0, _CHUNK, unroll=4)
                def _(j):
                    sb = plsc.load_gather(
                        s_v, [jnp.full((16,), j, jnp.int32)])
                    for k in range(d // 16):
                        sl = pl.ds(k * 16, 16)
                        rows_v[b, j, sl] = rows_v[b, j, sl] * sb

                pltpu.async_copy(rows_v.at[b], acc_sh.at[idx_v.at[q, 1]],
                                 ssems[b], add=True)

                @pl.when(c + 2 < _CPW)
                def _():
                    pltpu.async_copy(
                        eidx_hbm.at[wid, c + 2], idx_v.at[(q + 2) % 4],
                        isems[(q + 2) % 4])
            return 0

        lax.fori_loop(0, _CPW // 4, quad_body, 0)
        pltpu.make_async_copy(
            rows_v.at[(_CPW - 1) % 2],
            acc_sh.at[idx_v.at[(_CPW - 1) % 4, 1]],
            ssems[(_CPW - 1) % 2]).wait()
        plsc.subcore_barrier()
        for r0, nr in _ROW_PIECES:
            pltpu.sync_copy(acc_sh.at[pl.ds(sid * _RPT + r0, nr)],
                            out_hbm.at[cid, pl.ds(sid * _RPT + r0, nr)])

    return _agg


_agg128 = _make_agg_kernel(_D)


_RB = 1000


def _dinv_body(p0_ref, p1_ref, o_ref):
    deg = 1.0 + p0_ref[...] + p1_ref[...]
    o_ref[...] = lax.rsqrt(deg[:, 0:1])


def _mm1_body(x_ref, w_ref, o_ref):
    o_ref[...] = jnp.dot(x_ref[...], w_ref[...],
                         preferred_element_type=jnp.float32)


def _mm_mid_body(p0_ref, p1_ref, u_ref, dinv_ref, b_ref, w_ref, o_ref):
    dinv = dinv_ref[...]
    a = dinv * (p0_ref[...] + p1_ref[...] + dinv * u_ref[...]) + b_ref[...]
    h = jnp.where(a > 0, a, jnp.exp(jnp.minimum(a, 0.0)) - 1.0)
    o_ref[...] = jnp.dot(h, w_ref[...], preferred_element_type=jnp.float32)


def _final_body(p0_ref, p1_ref, u_ref, dinv_ref, b_ref, o_ref):
    dinv = dinv_ref[...]
    a = dinv * (p0_ref[...] + p1_ref[...] + dinv * u_ref[...]) + b_ref[...]
    col = lax.broadcasted_iota(jnp.int32, a.shape, 1)
    am = jnp.where(col < _DO, a, -jnp.inf)
    m = jnp.max(am, axis=1, keepdims=True)
    lse = jnp.log(jnp.sum(jnp.exp(am - m), axis=1, keepdims=True)) + m
    o_ref[...] = am - lse


def _row_spec(d):
    return pl.BlockSpec((_RB, d), lambda i: (i, 0))


def _full_spec(r, c):
    return pl.BlockSpec((r, c), lambda i: (0, 0))


def kernel(x, edge_index, edge_attr, W1, b1, W2, b2, W3, b3):
    src = edge_index[0]
    dst = edge_index[1]
    w = edge_attr

    pad = _EPAD - _E
    zpad_i = jnp.zeros((pad,), jnp.int32)
    src3 = jnp.concatenate([src, zpad_i]).reshape(_NW, _CPW, _CHUNK)
    dst3 = jnp.concatenate([dst, zpad_i]).reshape(_NW, _CPW, _CHUNK)
    w3 = jnp.concatenate([w, jnp.zeros((pad,), jnp.float32)]
                         ).reshape(_NW, _CPW, _CHUNK)
    eidx = jnp.stack([src3, dst3, lax.bitcast_convert_type(w3, jnp.int32),
                      jnp.zeros_like(src3)], axis=2)

    W3p = jnp.pad(W3, ((0, 0), (0, _DOP - _DO)))
    b3p = jnp.pad(b3, (0, _DOP - _DO)).reshape(1, _DOP)
    b1r = b1.reshape(1, _D)
    b2r = b2.reshape(1, _D)

    grid = (_N // _RB,)

    pdeg = _deg_kernel(dst3, w3.reshape(_NW, _EPW))
    u1 = pl.pallas_call(
        _mm1_body,
        grid=grid,
        in_specs=[_row_spec(_D), _full_spec(_D, _D)],
        out_specs=_row_spec(_D),
        out_shape=jax.ShapeDtypeStruct((_N, _D), jnp.float32),
    )(x, W1)

    dinv = pl.pallas_call(
        _dinv_body,
        grid=grid,
        in_specs=[_row_spec(16), _row_spec(16)],
        out_specs=_row_spec(1),
        out_shape=jax.ShapeDtypeStruct((_N, 1), jnp.float32),
    )(pdeg[0], pdeg[1])
    dinv_flat = dinv.reshape(_N)

    p1_ = _agg128(u1, dinv_flat, eidx)
    u2 = pl.pallas_call(
        _mm_mid_body,
        grid=grid,
        in_specs=[_row_spec(_D), _row_spec(_D), _row_spec(_D), _row_spec(1),
                  _full_spec(1, _D), _full_spec(_D, _D)],
        out_specs=_row_spec(_D),
        out_shape=jax.ShapeDtypeStruct((_N, _D), jnp.float32),
    )(p1_[0], p1_[1], u1, dinv, b1r, W2)

    p2_ = _agg128(u2, dinv_flat, eidx)
    u3 = pl.pallas_call(
        _mm_mid_body,
        grid=grid,
        in_specs=[_row_spec(_D), _row_spec(_D), _row_spec(_D), _row_spec(1),
                  _full_spec(1, _D), _full_spec(_D, _DOP)],
        out_specs=_row_spec(_DOP),
        out_shape=jax.ShapeDtypeStruct((_N, _DOP), jnp.float32),
    )(p2_[0], p2_[1], u2, dinv, b2r, W3p)

    p3_ = _agg128(u3, dinv_flat, eidx)
    out = pl.pallas_call(
        _final_body,
        grid=grid,
        in_specs=[_row_spec(_DOP), _row_spec(_DOP), _row_spec(_DOP),
                  _row_spec(1), _full_spec(1, _DOP)],
        out_specs=_row_spec(_DOP),
        out_shape=jax.ShapeDtypeStruct((_N, _DOP), jnp.float32),
    )(p3_[0], p3_[1], u3, dinv, b3p)
    return out[:, :_DO]


# speedup vs baseline: 7.9659x; 1.0195x over previous
import functools

import jax
import jax.numpy as jnp
from jax import lax
from jax.experimental import pallas as pl
from jax.experimental.pallas import tpu as pltpu
from jax.experimental.pallas import tpu_sc as plsc

_N = 10000
_E = 320000
_D = 128
_DO = 40
_DOP = 128

_NC = 2
_NS = 16
_NW = _NC * _NS
_CHUNK = 128
_CPW = 80
_EPW = _CPW * _CHUNK
_EPAD = _NW * _EPW
_NP = 10240
_RPT = _NP // _NS

_mesh = plsc.VectorSubcoreMesh(core_axis_name="c", subcore_axis_name="s")

_ROW_PIECES = tuple((i * _CHUNK, _CHUNK) for i in range(_RPT // _CHUNK))


def _zero_rows(buf, d):
    zero = jnp.zeros((16,), jnp.float32)

    def body(j, _):
        for k in range(d // 16):
            buf[j, pl.ds(k * 16, 16)] = zero
        return 0

    lax.fori_loop(0, _CHUNK, body, 0)


@functools.partial(
    pl.kernel,
    out_type=jax.ShapeDtypeStruct((_NC, _NP, 16), jnp.float32),
    mesh=_mesh,
    compiler_params=pltpu.CompilerParams(needs_layout_passes=False),
    scratch_types=[
        pltpu.VMEM((_CPW, _CHUNK), jnp.int32),
        pltpu.VMEM((_EPW,), jnp.float32),
        pltpu.VMEM((_CHUNK, 16), jnp.float32),
        pltpu.VMEM_SHARED((_NP, 16), jnp.float32),
    ],
)
def _deg_kernel(dst_hbm, w_hbm, out_hbm, dst_v, w_v, buf_v, acc_sh):
    cid = lax.axis_index("c")
    sid = lax.axis_index("s")
    wid = cid * _NS + sid
    pltpu.sync_copy(dst_hbm.at[wid], dst_v)
    pltpu.sync_copy(w_hbm.at[wid], w_v)
    _zero_rows(buf_v, 16)
    for r0, nr in _ROW_PIECES:
        pltpu.sync_copy(buf_v.at[pl.ds(0, nr)],
                        acc_sh.at[pl.ds(sid * _RPT + r0, nr)])
    plsc.subcore_barrier()

    def chunk_body(c, _):
        @plsc.parallel_loop(0, _CHUNK, unroll=4)
        def _(j):
            wj = plsc.load_gather(
                w_v, [jnp.full((16,), c * _CHUNK + j, jnp.int32)])
            buf_v[j, :] = wj
        pltpu.sync_copy(buf_v, acc_sh.at[dst_v.at[c]], add=True)
        return 0

    lax.fori_loop(0, _CPW, chunk_body, 0)
    plsc.subcore_barrier()
    for r0, nr in _ROW_PIECES:
        pltpu.sync_copy(acc_sh.at[pl.ds(sid * _RPT + r0, nr)],
                        out_hbm.at[cid, pl.ds(sid * _RPT + r0, nr)])


def _make_agg_kernel(d):
    @functools.partial(
        pl.kernel,
        out_type=jax.ShapeDtypeStruct((_NC, _NP, d), jnp.float32),
        mesh=_mesh,
        compiler_params=pltpu.CompilerParams(needs_layout_passes=False),
        scratch_types=[
            pltpu.VMEM((4, 4, _CHUNK), jnp.int32),
            pltpu.VMEM((_N,), jnp.float32),
            pltpu.VMEM((_CHUNK,), jnp.float32),
            pltpu.VMEM((2, _CHUNK, d), jnp.float32),
            pltpu.VMEM_SHARED((_NP, d), jnp.float32),
            pltpu.SemaphoreType.DMA,
            pltpu.SemaphoreType.DMA,
            pltpu.SemaphoreType.DMA,
            pltpu.SemaphoreType.DMA,
            pltpu.SemaphoreType.DMA,
            pltpu.SemaphoreType.DMA,
            pltpu.SemaphoreType.DMA,
            pltpu.SemaphoreType.DMA,
        ],
    )
    def _agg(u_hbm, dinv_hbm, eidx_hbm, out_hbm,
             idx_v, dinv_v, s_v, rows_v, acc_sh,
             gs0, gs1, ss0, ss1, is0, is1, is2, is3):
        gsems = (gs0, gs1)
        ssems = (ss0, ss1)
        isems = (is0, is1, is2, is3)
        cid = lax.axis_index("c")
        sid = lax.axis_index("s")
        wid = cid * _NS + sid
        pltpu.sync_copy(dinv_hbm, dinv_v)
        _zero_rows(rows_v.at[0], d)
        for r0, nr in _ROW_PIECES:
            pltpu.sync_copy(rows_v.at[0, pl.ds(0, nr)],
                            acc_sh.at[pl.ds(sid * _RPT + r0, nr)])
        plsc.subcore_barrier()

        for q in range(2):
            pltpu.async_copy(eidx_hbm.at[wid, q], idx_v.at[q], isems[q])
        pltpu.make_async_copy(eidx_hbm.at[wid, 0], idx_v.at[0], is0).wait()
        pltpu.async_copy(u_hbm.at[idx_v.at[0, 0]], rows_v.at[0], gs0)

        def quad_body(i, _):
            for b4 in range(4):
                c = 4 * i + b4
                b = b4 % 2
                nb = 1 - b
                q = b4
                nq = (b4 + 1) % 4

                @pl.when(c >= 1)
                def _():
                    pltpu.make_async_copy(
                        rows_v.at[nb],
                        acc_sh.at[idx_v.at[(q + 3) % 4, 1]],
                        ssems[nb]).wait()

                @pl.when(c + 1 < _CPW)
                def _():
                    pltpu.make_async_copy(
                        eidx_hbm.at[wid, c + 1], idx_v.at[nq],
                        isems[nq]).wait()
                    pltpu.async_copy(
                        u_hbm.at[idx_v.at[nq, 0]], rows_v.at[nb], gsems[nb])

                pltpu.make_async_copy(
                    u_hbm.at[idx_v.at[q, 0]], rows_v.at[b], gsems[b]).wait()

                @plsc.parallel_loop(0, _CHUNK // 16, unroll=2)
                def _(t):
                    sl = pl.ds(t * 16, 16)
                    src16 = idx_v[q, 0, sl]
                    w16 = plsc.bitcast(idx_v[q, 2, sl], jnp.float32)
                    s_v[sl] = w16 * plsc.load_gather(dinv_v, [src16])

                @plsc.parallel_loop(0, _CHUNK, unroll=4)
                def _(j):
                    sb = plsc.load_gather(
                        s_v, [jnp.full((16,), j, jnp.int32)])
                    for k in range(d // 16):
                        sl = pl.ds(k * 16, 16)
                        rows_v[b, j, sl] = rows_v[b, j, sl] * sb

                pltpu.async_copy(rows_v.at[b], acc_sh.at[idx_v.at[q, 1]],
                                 ssems[b], add=True)

                @pl.when(c + 2 < _CPW)
                def _():
                    pltpu.async_copy(
                        eidx_hbm.at[wid, c + 2], idx_v.at[(q + 2) % 4],
                        isems[(q + 2) % 4])
            return 0

        lax.fori_loop(0, _CPW // 4, quad_body, 0)
        pltpu.make_async_copy(
            rows_v.at[(_CPW - 1) % 2],
            acc_sh.at[idx_v.at[(_CPW - 1) % 4, 1]],
            ssems[(_CPW - 1) % 2]).wait()
        plsc.subcore_barrier()
        for r0, nr in _ROW_PIECES:
            pltpu.sync_copy(acc_sh.at[pl.ds(sid * _RPT + r0, nr)],
                            out_hbm.at[cid, pl.ds(sid * _RPT + r0, nr)])

    return _agg


_agg128 = _make_agg_kernel(_D)


_RB = 1000


def _dinv_body(p0_ref, p1_ref, o_ref):
    deg = 1.0 + p0_ref[...] + p1_ref[...]
    o_ref[...] = lax.rsqrt(deg[:, 0:1])


def _mm1_body(x_ref, w_ref, o_ref):
    o_ref[...] = jnp.dot(x_ref[...], w_ref[...],
                         preferred_element_type=jnp.float32)


def _mm_mid_body(p0_ref, p1_ref, u_ref, dinv_ref, b_ref, w_ref, o_ref):
    dinv = dinv_ref[...]
    a = dinv * (p0_ref[...] + p1_ref[...] + dinv * u_ref[...]) + b_ref[...]
    h = jnp.where(a > 0, a, jnp.exp(jnp.minimum(a, 0.0)) - 1.0)
    o_ref[...] = jnp.dot(h, w_ref[...], preferred_element_type=jnp.float32)


def _final_body(p0_ref, p1_ref, u_ref, dinv_ref, b_ref, o_ref):
    dinv = dinv_ref[...]
    a = dinv * (p0_ref[...] + p1_ref[...] + dinv * u_ref[...]) + b_ref[...]
    col = lax.broadcasted_iota(jnp.int32, a.shape, 1)
    am = jnp.where(col < _DO, a, -jnp.inf)
    m = jnp.max(am, axis=1, keepdims=True)
    lse = jnp.log(jnp.sum(jnp.exp(am - m), axis=1, keepdims=True)) + m
    o_ref[...] = am - lse


def _row_spec(d):
    return pl.BlockSpec((_RB, d), lambda i: (i, 0))


def _full_spec(r, c):
    return pl.BlockSpec((r, c), lambda i: (0, 0))


def kernel(x, edge_index, edge_attr, W1, b1, W2, b2, W3, b3):
    src = edge_index[0]
    dst = edge_index[1]
    w = edge_attr

    pad = _EPAD - _E
    zpad_i = jnp.zeros((pad,), jnp.int32)
    src3 = jnp.concatenate([src, zpad_i]).reshape(_NW, _CPW, _CHUNK)
    dst3 = jnp.concatenate([dst, zpad_i]).reshape(_NW, _CPW, _CHUNK)
    w3 = jnp.concatenate([w, jnp.zeros((pad,), jnp.float32)]
                         ).reshape(_NW, _CPW, _CHUNK)
    eidx = jnp.stack([src3, dst3, lax.bitcast_convert_type(w3, jnp.int32),
                      jnp.zeros_like(src3)], axis=2)

    W3p = jnp.pad(W3, ((0, 0), (0, _DOP - _DO)))
    b3p = jnp.pad(b3, (0, _DOP - _DO)).reshape(1, _DOP)
    b1r = b1.reshape(1, _D)
    b2r = b2.reshape(1, _D)

    grid = (_N // _RB,)

    pdeg = _deg_kernel(dst3, w3.reshape(_NW, _EPW))
    u1 = pl.pallas_call(
        _mm1_body,
        grid=grid,
        in_specs=[_row_spec(_D), _full_spec(_D, _D)],
        out_specs=_row_spec(_D),
        out_shape=jax.ShapeDtypeStruct((_N, _D), jnp.float32),
    )(x, W1)

    dinv = pl.pallas_call(
        _dinv_body,
        grid=grid,
        in_specs=[_row_spec(16), _row_spec(16)],
        out_specs=_row_spec(1),
        out_shape=jax.ShapeDtypeStruct((_N, 1), jnp.float32),
    )(pdeg[0], pdeg[1])
    dinv_flat = dinv.reshape(_N)

    p1_ = _agg128(u1, dinv_flat, eidx)
    u2 = pl.pallas_call(
        _mm_mid_body,
        grid=grid,
        in_specs=[_row_spec(_D), _row_spec(_D), _row_spec(_D), _row_spec(1),
                  _full_spec(1, _D), _full_spec(_D, _D)],
        out_specs=_row_spec(_D),
        out_shape=jax.ShapeDtypeStruct((_N, _D), jnp.float32),
    )(p1_[0], p1_[1], u1, dinv, b1r, W2)

    p2_ = _agg128(u2, dinv_flat, eidx)
    u3 = pl.pallas_call(
        _mm_mid_body,
        grid=grid,
        in_specs=[_row_spec(_D), _row_spec(_D), _row_spec(_D), _row_spec(1),
                  _full_spec(1, _D), _full_spec(_D, _DOP)],
        out_specs=_row_spec(_DOP),
        out_shape=jax.ShapeDtypeStruct((_N, _DOP), jnp.float32),
    )(p2_[0], p2_[1], u2, dinv, b2r, W3p)

    p3_ = _agg128(u3, dinv_flat, eidx)
    out = pl.pallas_call(
        _final_body,
        grid=grid,
        in_specs=[_row_spec(_DOP), _row_spec(_DOP), _row_spec(_DOP),
                  _row_spec(1), _full_spec(1, _DOP)],
        out_specs=_row_spec(_DOP),
        out_shape=jax.ShapeDtypeStruct((_N, _DOP), jnp.float32),
    )(p3_[0], p3_[1], u3, dinv, b3p)
    return out[:, :_DO]
